# Initial kernel scaffold; baseline (speedup 1.0000x reference)
import jax
import jax.numpy as jnp
from jax.experimental import pallas as pl


def kernel(x, attention_mask):
    raise NotImplementedError("write your pallas kernel here")


# speedup vs baseline: 2.6558x; 2.6558x over previous
import jax
import jax.numpy as jnp
from jax.experimental import pallas as pl

_SBLK = 1024


def _pool_body(mask_ref, x_ref, out_ref):
    b = pl.program_id(0)
    n = pl.program_id(1)
    nblocks = pl.num_programs(1)

    mask_row = mask_ref[pl.ds(b, 1), :]
    length = jnp.sum(mask_row)
    sep = length.astype(jnp.int32) - 1

    sblk = x_ref.shape[1]
    ids = jax.lax.broadcasted_iota(jnp.int32, (1, sblk), 1) + n * sblk
    w_blk = mask_ref[pl.ds(b, 1), pl.ds(n * sblk, sblk)]
    keep = (ids != 0) & (ids != sep)
    w = w_blk * keep.astype(jnp.float32)

    contrib = jnp.dot(w, x_ref[0], preferred_element_type=jnp.float32)

    @pl.when(n == 0)
    def _init():
        out_ref[...] = jnp.zeros_like(out_ref)

    out_ref[0] += contrib

    @pl.when(n == nblocks - 1)
    def _finalize():
        out_ref[...] = out_ref[...] / (length - 2.0)


def kernel(x, attention_mask):
    B, S, D = x.shape
    n = S // _SBLK
    out = pl.pallas_call(
        _pool_body,
        grid=(B, n),
        in_specs=[
            pl.BlockSpec((B, S), lambda b, i: (0, 0)),
            pl.BlockSpec((1, _SBLK, D), lambda b, i: (b, i, 0)),
        ],
        out_specs=pl.BlockSpec((1, 1, D), lambda b, i: (b, 0, 0)),
        out_shape=jax.ShapeDtypeStruct((B, 1, D), jnp.float32),
    )(attention_mask, x)
    return out.reshape(B, D)


# speedup vs baseline: 3.0713x; 1.1564x over previous
import jax
import jax.numpy as jnp
from jax.experimental import pallas as pl

_SBLK = 2048


def _pool_body(mask_ref, x_ref, out_ref):
    b = pl.program_id(0)
    n = pl.program_id(1)
    nblocks = pl.num_programs(1)

    mask_row = mask_ref[pl.ds(b, 1), :]
    length = jnp.sum(mask_row)
    sep = length.astype(jnp.int32) - 1

    sblk = x_ref.shape[1]
    ids = jax.lax.broadcasted_iota(jnp.int32, (1, sblk), 1) + n * sblk
    w_blk = mask_ref[pl.ds(b, 1), pl.ds(n * sblk, sblk)]
    keep = (ids != 0) & (ids != sep)
    w = w_blk * keep.astype(jnp.float32)

    contrib = jnp.dot(w, x_ref[0], preferred_element_type=jnp.float32)

    @pl.when(n == 0)
    def _init():
        out_ref[...] = jnp.zeros_like(out_ref)

    out_ref[0] += contrib

    @pl.when(n == nblocks - 1)
    def _finalize():
        out_ref[...] = out_ref[...] / (length - 2.0)


def kernel(x, attention_mask):
    B, S, D = x.shape
    n = S // _SBLK
    out = pl.pallas_call(
        _pool_body,
        grid=(B, n),
        in_specs=[
            pl.BlockSpec((B, S), lambda b, i: (0, 0)),
            pl.BlockSpec((1, _SBLK, D), lambda b, i: (b, i, 0)),
        ],
        out_specs=pl.BlockSpec((1, 1, D), lambda b, i: (b, 0, 0)),
        out_shape=jax.ShapeDtypeStruct((B, 1, D), jnp.float32),
    )(attention_mask, x)
    return out.reshape(B, D)
